# Initial kernel scaffold; baseline (speedup 1.0000x reference)
import jax
import jax.numpy as jnp
from jax.experimental import pallas as pl


def kernel(x, xyz, nbr_idx, W_pre, b_pre, W1, b1, W2, b2, W3, b3, W4, b4):
    raise NotImplementedError("write your pallas kernel here")


# speedup vs baseline: 1.1996x; 1.1996x over previous
import functools

import jax
import jax.numpy as jnp
from jax import lax
from jax.experimental import pallas as pl
from jax.experimental.pallas import tpu as pltpu
from jax.experimental.pallas import tpu_sc as plsc

GAMMA = 10.0
N, K, NF = 10000, 16, 300
LN2 = 0.6931471805599453

NC, NS = 2, 16
NW = NC * NS
NPAD = 10240
B_EDGES = NPAD * K
CHUNK = 128
NCHUNK = B_EDGES // (NW * CHUNK)
INNER = 8
OUTER = NCHUNK // INNER


def _sc_gather(table, idx3):
    mesh = plsc.VectorSubcoreMesh(core_axis_name="c", subcore_axis_name="s")

    @functools.partial(
        pl.kernel,
        mesh=mesh,
        out_type=jax.ShapeDtypeStruct((NW, NCHUNK, CHUNK, 16), jnp.float32),
        scratch_types=[
            pltpu.VMEM((NCHUNK, CHUNK), jnp.int32),
            pltpu.VMEM((NCHUNK, CHUNK, 16), jnp.float32),
            pltpu.SemaphoreType.DMA,
        ],
        compiler_params=pltpu.CompilerParams(use_tc_tiling_on_sc=False),
    )
    def k(table_hbm, idx_hbm, out_hbm, idx_v, rows_v, sem):
        wid = lax.axis_index("s") * NC + lax.axis_index("c")
        pltpu.sync_copy(idx_hbm.at[wid], idx_v)

        def body(g, carry):
            handles = []
            for b in range(INNER):
                j = g * INNER + b
                handles.append(
                    pltpu.async_copy(table_hbm.at[idx_v.at[j]],
                                     rows_v.at[j], sem))
            for h in handles:
                h.wait()
            return carry

        lax.fori_loop(0, OUTER, body, 0)
        pltpu.sync_copy(rows_v, out_hbm.at[wid])

    return k(table, idx3)


def _tc_body(x_ref, src_ref, own_ref, cen_ref,
             wp_ref, bp_ref, w1_ref, b1_ref, w2_ref, b2_ref,
             w3_ref, b3_ref, w4_ref, b4_ref, out_ref, *, bn):
    def ssp(v):
        return (jnp.maximum(v, 0.0)
                + jnp.log1p(jnp.exp(-jnp.abs(v))) - LN2)

    x = x_ref[...]
    src = src_ref[...]
    own = own_ref[...]
    own_e = jnp.broadcast_to(own[:, None, :], (bn, K, 16)).reshape(bn * K, 16)
    diff = src - own_e
    d2 = jnp.sum(diff * diff, axis=1, keepdims=True)
    d = jnp.sqrt(d2 + 1e-12)
    t = d - cen_ref[...]
    rbf = jnp.exp(-GAMMA * (t * t))
    h = ssp(jnp.dot(rbf, w1_ref[...],
                    preferred_element_type=jnp.float32) + b1_ref[...])
    conv = ssp(jnp.dot(h, w2_ref[...],
                       preferred_element_type=jnp.float32) + b2_ref[...])
    s = jnp.sum(conv.reshape(bn, K, NF), axis=1)
    pre = jnp.dot(x, wp_ref[...],
                  preferred_element_type=jnp.float32) + bp_ref[...]
    msg = pre * s
    post = jnp.dot(ssp(jnp.dot(msg, w3_ref[...],
                               preferred_element_type=jnp.float32)
                       + b3_ref[...]),
                   w4_ref[...], preferred_element_type=jnp.float32)
    out_ref[...] = x + post + b4_ref[...]


def _tc_main(x_pad, src, own, cen, wp, bp, w1, b1, w2, b2, w3, b3, w4, b4,
             bn=128):
    grid = NPAD // bn
    full = lambda i: (0, 0)
    return pl.pallas_call(
        functools.partial(_tc_body, bn=bn),
        grid=(grid,),
        in_specs=[
            pl.BlockSpec((bn, NF), lambda i: (i, 0)),
            pl.BlockSpec((bn * K, 16), lambda i: (i, 0)),
            pl.BlockSpec((bn, 16), lambda i: (i, 0)),
            pl.BlockSpec((1, NF), full),
            pl.BlockSpec((NF, NF), full),
            pl.BlockSpec((1, NF), full),
            pl.BlockSpec((NF, NF), full),
            pl.BlockSpec((1, NF), full),
            pl.BlockSpec((NF, NF), full),
            pl.BlockSpec((1, NF), full),
            pl.BlockSpec((NF, NF), full),
            pl.BlockSpec((1, NF), full),
            pl.BlockSpec((NF, NF), full),
            pl.BlockSpec((1, NF), full),
        ],
        out_specs=pl.BlockSpec((bn, NF), lambda i: (i, 0)),
        out_shape=jax.ShapeDtypeStruct((NPAD, NF), jnp.float32),
        compiler_params=pltpu.CompilerParams(
            dimension_semantics=("arbitrary",)),
    )(x_pad, src, own, cen, wp, bp, w1, b1, w2, b2, w3, b3, w4, b4)


def kernel(x, xyz, nbr_idx, W_pre, b_pre, W1, b1, W2, b2, W3, b3, W4, b4):
    table = jnp.pad(xyz.astype(jnp.float32), ((0, 0), (0, 13)))
    idx = jnp.pad(nbr_idx.astype(jnp.int32).reshape(-1),
                  (0, B_EDGES - N * K)).reshape(NW, NCHUNK, CHUNK)
    src = _sc_gather(table, idx).reshape(NPAD * K, 16)
    own = jnp.pad(table, ((0, NPAD - N), (0, 0)))
    x_pad = jnp.pad(x, ((0, NPAD - N), (0, 0)))
    cen = jnp.linspace(0.1, 30.1, NF).astype(jnp.float32).reshape(1, NF)
    out = _tc_main(x_pad, src, own, cen,
                   W_pre, b_pre.reshape(1, NF),
                   W1, b1.reshape(1, NF), W2, b2.reshape(1, NF),
                   W3, b3.reshape(1, NF), W4, b4.reshape(1, NF))
    return out[:N]


# speedup vs baseline: 1.5405x; 1.2843x over previous
import functools

import jax
import jax.numpy as jnp
from jax import lax
from jax.experimental import pallas as pl
from jax.experimental.pallas import tpu as pltpu
from jax.experimental.pallas import tpu_sc as plsc

GAMMA = 10.0
N, K, NF = 10000, 16, 300
LN2 = 0.6931471805599453

NC, NS = 2, 16
NW = NC * NS
B_EDGES = N * K
CHUNK = 125
NCHUNK = B_EDGES // (NW * CHUNK)
INNER = 8
OUTER = NCHUNK // INNER


def _sc_gather(table, idx3):
    mesh = plsc.VectorSubcoreMesh(core_axis_name="c", subcore_axis_name="s")

    @functools.partial(
        pl.kernel,
        mesh=mesh,
        out_type=jax.ShapeDtypeStruct((NW, NCHUNK, CHUNK, 16), jnp.float32),
        scratch_types=[
            pltpu.VMEM((NCHUNK, CHUNK), jnp.int32),
            pltpu.VMEM((NCHUNK, CHUNK, 16), jnp.float32),
            pltpu.SemaphoreType.DMA,
        ],
        compiler_params=pltpu.CompilerParams(use_tc_tiling_on_sc=False),
    )
    def k(table_hbm, idx_hbm, out_hbm, idx_v, rows_v, sem):
        wid = lax.axis_index("s") * NC + lax.axis_index("c")
        pltpu.sync_copy(idx_hbm.at[wid], idx_v)

        def body(g, carry):
            handles = []
            for b in range(INNER):
                j = g * INNER + b
                handles.append(
                    pltpu.async_copy(table_hbm.at[idx_v.at[j]],
                                     rows_v.at[j], sem))
            for h in handles:
                h.wait()
            return carry

        lax.fori_loop(0, OUTER, body, 0)
        pltpu.sync_copy(rows_v, out_hbm.at[wid])

    return k(table, idx3)


def _tc_body(x_ref, src_ref, own_ref, cen_ref,
             wp_ref, bp_ref, w1_ref, b1_ref, w2_ref, b2_ref,
             w3_ref, b3_ref, w4_ref, b4_ref, out_ref, *, bn):
    def ssp(v):
        return (jnp.maximum(v, 0.0)
                + jnp.log1p(jnp.exp(-jnp.abs(v))) - LN2)

    x = x_ref[...]
    src = src_ref[...]
    own = own_ref[...]
    own_e = jnp.broadcast_to(own[:, None, :], (bn, K, 16)).reshape(bn * K, 16)
    diff = src - own_e
    d2 = jnp.sum(diff * diff, axis=1, keepdims=True)
    d = jnp.sqrt(d2 + 1e-12)
    t = d - cen_ref[...]
    rbf = jnp.exp(-GAMMA * (t * t))
    h = ssp(jnp.dot(rbf, w1_ref[...],
                    preferred_element_type=jnp.float32) + b1_ref[...])
    conv = ssp(jnp.dot(h, w2_ref[...],
                       preferred_element_type=jnp.float32) + b2_ref[...])
    s = jnp.sum(conv.reshape(bn, K, NF), axis=1)
    pre = jnp.dot(x, wp_ref[...],
                  preferred_element_type=jnp.float32) + bp_ref[...]
    msg = pre * s
    post = jnp.dot(ssp(jnp.dot(msg, w3_ref[...],
                               preferred_element_type=jnp.float32)
                       + b3_ref[...]),
                   w4_ref[...], preferred_element_type=jnp.float32)
    out_ref[...] = x + post + b4_ref[...]


def _tc_main(x, src, own, cen, wp, bp, w1, b1, w2, b2, w3, b3, w4, b4,
             bn=200):
    grid = N // bn
    full = lambda i: (0, 0)
    return pl.pallas_call(
        functools.partial(_tc_body, bn=bn),
        grid=(grid,),
        in_specs=[
            pl.BlockSpec((bn, NF), lambda i: (i, 0)),
            pl.BlockSpec((bn * K, 16), lambda i: (i, 0)),
            pl.BlockSpec((bn, 16), lambda i: (i, 0)),
            pl.BlockSpec((1, NF), full),
            pl.BlockSpec((NF, NF), full),
            pl.BlockSpec((1, NF), full),
            pl.BlockSpec((NF, NF), full),
            pl.BlockSpec((1, NF), full),
            pl.BlockSpec((NF, NF), full),
            pl.BlockSpec((1, NF), full),
            pl.BlockSpec((NF, NF), full),
            pl.BlockSpec((1, NF), full),
            pl.BlockSpec((NF, NF), full),
            pl.BlockSpec((1, NF), full),
        ],
        out_specs=pl.BlockSpec((bn, NF), lambda i: (i, 0)),
        out_shape=jax.ShapeDtypeStruct((N, NF), jnp.float32),
        compiler_params=pltpu.CompilerParams(
            dimension_semantics=("arbitrary",)),
    )(x, src, own, cen, wp, bp, w1, b1, w2, b2, w3, b3, w4, b4)


def kernel(x, xyz, nbr_idx, W_pre, b_pre, W1, b1, W2, b2, W3, b3, W4, b4):
    table = jnp.pad(xyz.astype(jnp.float32), ((0, 0), (0, 13)))
    idx = nbr_idx.astype(jnp.int32).reshape(NW, NCHUNK, CHUNK)
    src = _sc_gather(table, idx).reshape(B_EDGES, 16)
    cen = jnp.linspace(0.1, 30.1, NF).astype(jnp.float32).reshape(1, NF)
    return _tc_main(x, src, table, cen,
                    W_pre, b_pre.reshape(1, NF),
                    W1, b1.reshape(1, NF), W2, b2.reshape(1, NF),
                    W3, b3.reshape(1, NF), W4, b4.reshape(1, NF))
